# baseline (speedup 1.0000x reference)
import functools

import jax
import jax.numpy as jnp
from jax import lax
from jax.experimental import pallas as pl
from jax.experimental.pallas import tpu as pltpu
from jax.experimental.pallas import tpu_sc as plsc

N = 10000
E = 320000
D = 128
H = 64
OUT = 64
G = 64
EPS = 1e-5
W2 = 2 * H

NC = 2
NS = 16
NW = NC * NS
EW = E // NW
K = 80
NCHUNK = EW // K
TE = 3200
NP = 10240
NROWP = NP // NS
ZR = 64


def _k1_body(x_ref, g0_ref, b0_ref, We1_ref, Wn1a_ref,
             xbn_ref, TR_ref, TB_ref):
    xv = x_ref[...]
    m = jnp.mean(xv, axis=0, keepdims=True)
    xc = xv - m
    v = jnp.mean(xc * xc, axis=0, keepdims=True)
    xbn = xc * (1.0 / jnp.sqrt(v + EPS)) * g0_ref[...] + b0_ref[...]
    xbn_ref[...] = xbn
    A = jnp.dot(xbn, We1_ref[:D, :], preferred_element_type=jnp.float32)
    B = jnp.dot(xbn, We1_ref[D:, :], preferred_element_type=jnp.float32)
    C = jnp.dot(xbn, Wn1a_ref[:D, :], preferred_element_type=jnp.float32)
    TR_ref[...] = jnp.concatenate([A, C], axis=1)
    TB_ref[...] = jnp.concatenate([B, jnp.zeros_like(B)], axis=1)


def _k2_body(TR_hbm, TB_hbm, row3_hbm, col3_hbm,
             t_out, stat_out,
             idxR, idxC, bufR0, bufB0, bufR1, bufB1, stats,
             semR0, semB0, semR1, semB1, semW0, semW1):
    wid = lax.axis_index("s") * NC + lax.axis_index("c")
    base_e = wid * EW

    pltpu.sync_copy(row3_hbm.at[wid], idxR)
    pltpu.sync_copy(col3_hbm.at[wid], idxC)

    def fire(ci, bufR, bufB, semR, semB):
        pltpu.async_copy(TR_hbm.at[idxR.at[ci]], bufR, semR)
        pltpu.async_copy(TB_hbm.at[idxC.at[ci]], bufB, semB)

    def drain_w(bufR, semW):
        pltpu.make_async_copy(bufR, t_out.at[pl.ds(base_e, K)], semW).wait()

    def process(ci, bufR, bufB, semR, semB, semW, carry):
        pltpu.make_async_copy(TR_hbm.at[idxR.at[0]], bufR, semR).wait()
        pltpu.make_async_copy(TB_hbm.at[idxC.at[0]], bufB, semB).wait()

        def row_body(r, c):
            s0, s1, s2, s3, q0, q1, q2, q3 = c
            a0 = bufR[r, pl.ds(0, 16)]
            t0 = a0 + bufB[r, pl.ds(0, 16)]
            bufR[r, pl.ds(0, 16)] = t0
            a1 = bufR[r, pl.ds(16, 16)]
            t1 = a1 + bufB[r, pl.ds(16, 16)]
            bufR[r, pl.ds(16, 16)] = t1
            a2 = bufR[r, pl.ds(32, 16)]
            t2 = a2 + bufB[r, pl.ds(32, 16)]
            bufR[r, pl.ds(32, 16)] = t2
            a3 = bufR[r, pl.ds(48, 16)]
            t3 = a3 + bufB[r, pl.ds(48, 16)]
            bufR[r, pl.ds(48, 16)] = t3
            return (s0 + t0, s1 + t1, s2 + t2, s3 + t3,
                    q0 + t0 * t0, q1 + t1 * t1, q2 + t2 * t2, q3 + t3 * t3)

        carry = lax.fori_loop(0, K, row_body, carry)
        pltpu.async_copy(bufR, t_out.at[pl.ds(base_e + ci * K, K)], semW)
        return carry

    fire(0, bufR0, bufB0, semR0, semB0)
    z16 = jnp.zeros((16,), jnp.float32)
    carry0 = (z16,) * 8

    def pair(pi, carry):
        ci0 = 2 * pi

        @pl.when(pi > 0)
        def _():
            drain_w(bufR1, semW1)

        fire(ci0 + 1, bufR1, bufB1, semR1, semB1)
        carry = process(ci0, bufR0, bufB0, semR0, semB0, semW0, carry)
        drain_w(bufR0, semW0)
        fire(ci0 + 2, bufR0, bufB0, semR0, semB0)
        carry = process(ci0 + 1, bufR1, bufB1, semR1, semB1, semW1, carry)
        return carry

    carry = lax.fori_loop(0, (NCHUNK - 1) // 2, pair, carry0)
    carry = process(NCHUNK - 1, bufR0, bufB0, semR0, semB0, semW0, carry)
    drain_w(bufR0, semW0)
    drain_w(bufR1, semW1)
    s0, s1, s2, s3, q0, q1, q2, q3 = carry
    stats[pl.ds(0, 16)] = s0
    stats[pl.ds(16, 16)] = s1
    stats[pl.ds(32, 16)] = s2
    stats[pl.ds(48, 16)] = s3
    stats[pl.ds(64, 16)] = q0
    stats[pl.ds(80, 16)] = q1
    stats[pl.ds(96, 16)] = q2
    stats[pl.ds(112, 16)] = q3
    pltpu.sync_copy(stats, stat_out.at[wid])


def _k3_body(t_ref, stat_ref, We2_ref, be2_ref, Wn1ab_ref, bn1a_ref,
             ge1_ref, bbe1_ref, gn1_ref, bbn1_ref, Wn1b_ref, bn1b_ref,
             h3p_ref, acc_ref):
    p = pl.program_id(0)
    i = pl.program_id(1)

    ssum = jnp.sum(stat_ref[...], axis=0, keepdims=True)
    m1 = ssum[:, :H] / E
    v1 = ssum[:, H:] / E - m1 * m1
    s1 = ge1_ref[...] * (1.0 / jnp.sqrt(v1 + EPS))
    sh1 = bbe1_ref[...] - m1 * s1

    blk = t_ref[...]
    act = jnp.maximum(blk[:, :H] * s1 + sh1, 0.0)
    e2 = jnp.dot(act, We2_ref[...], preferred_element_type=jnp.float32) + be2_ref[...]
    hin = (jnp.dot(e2, Wn1ab_ref[...], preferred_element_type=jnp.float32)
           + bn1a_ref[...] + blk[:, H:])

    @pl.when((p == 0) & (i == 0))
    def _():
        acc_ref[...] = jnp.zeros_like(acc_ref)

    @pl.when(p == 0)
    def _():
        ps = jnp.sum(hin, axis=0, keepdims=True)
        ps2 = jnp.sum(hin * hin, axis=0, keepdims=True)
        acc_ref[...] += jnp.concatenate([ps, ps2], axis=1)

    @pl.when(p == 1)
    def _():
        tot = acc_ref[...]
        m2 = tot[:, :H] / E
        v2 = tot[:, H:] / E - m2 * m2
        s2 = gn1_ref[...] * (1.0 / jnp.sqrt(v2 + EPS))
        sh2 = bbn1_ref[...] - m2 * s2
        r = jnp.maximum(hin * s2 + sh2, 0.0)
        h3 = jnp.dot(r, Wn1b_ref[...], preferred_element_type=jnp.float32) + bn1b_ref[...]
        lane = lax.broadcasted_iota(jnp.int32, (TE, H), 1)
        cl = jnp.where(lane == 0, 1.0, 0.0)
        h3p_ref[...] = jnp.concatenate([h3, cl], axis=1)


def _k4_body(hp_hbm, col3_hbm,
             acc_out,
             idxC, hbuf0, hbuf1, zbuf,
             acc_sp,
             semH0, semH1, semS0, semS1):
    sid = lax.axis_index("s")
    cid = lax.axis_index("c")
    wid = sid * NC + cid
    base_e = wid * EW

    zero16 = jnp.zeros((16,), jnp.float32)

    def zb_body(r, _):
        for j in range(W2 // 16):
            zbuf[r, pl.ds(j * 16, 16)] = zero16
        return 0

    lax.fori_loop(0, ZR, zb_body, 0)

    for k in range(NROWP // ZR):
        r0 = sid * NROWP + k * ZR
        pltpu.sync_copy(zbuf, acc_sp.at[pl.ds(r0, ZR)])
    plsc.subcore_barrier()

    pltpu.sync_copy(col3_hbm.at[wid], idxC)

    def fire_read(ci, hbuf, semH):
        pltpu.async_copy(hp_hbm.at[pl.ds(base_e + ci * K, K)], hbuf, semH)

    def drain_s(hbuf, semS):
        pltpu.make_async_copy(hbuf, acc_sp.at[idxC.at[0]], semS).wait()

    def process(ci, hbuf, semH, semS):
        pltpu.make_async_copy(hp_hbm.at[pl.ds(base_e, K)], hbuf, semH).wait()
        pltpu.async_copy(hbuf, acc_sp.at[idxC.at[ci]], semS, add=True)

    fire_read(0, hbuf0, semH0)

    def pair(pi, _):
        ci0 = 2 * pi

        @pl.when(pi > 0)
        def _():
            drain_s(hbuf1, semS1)

        fire_read(ci0 + 1, hbuf1, semH1)
        process(ci0, hbuf0, semH0, semS0)
        drain_s(hbuf0, semS0)
        fire_read(ci0 + 2, hbuf0, semH0)
        process(ci0 + 1, hbuf1, semH1, semS1)
        return 0

    lax.fori_loop(0, (NCHUNK - 1) // 2, pair, 0)
    process(NCHUNK - 1, hbuf0, semH0, semS0)
    drain_s(hbuf0, semS0)
    drain_s(hbuf1, semS1)
    plsc.subcore_barrier()

    for k in range(NROWP // ZR):
        r0 = sid * NROWP + k * ZR
        pltpu.sync_copy(acc_sp.at[pl.ds(r0, ZR)], zbuf)
        pltpu.sync_copy(zbuf, acc_out.at[cid, pl.ds(r0, ZR)])


def _k5_body(xbn_ref, acc0_ref, acc1_ref, batch_ref,
             Wn2a_t_ref, Wn2a_b_ref, bn2a_ref,
             gn2_ref, bbn2_ref, Wn2b_ref, bn2b_ref,
             Wg1_ref, bg1_ref, gg1_ref, bbg1_ref, Wg2_ref, bg2_ref,
             un_ref):
    tot = acc0_ref[...] + acc1_ref[...]
    acc = tot[:N, :H]
    cnt = jnp.sum(tot[:N, H:], axis=1, keepdims=True)
    agg = acc / jnp.maximum(cnt, 1.0)
    h2 = (jnp.dot(xbn_ref[...], Wn2a_t_ref[...], preferred_element_type=jnp.float32)
          + jnp.dot(agg, Wn2a_b_ref[...], preferred_element_type=jnp.float32)
          + bn2a_ref[...])
    m = jnp.mean(h2, axis=0, keepdims=True)
    hc = h2 - m
    v = jnp.mean(hc * hc, axis=0, keepdims=True)
    h2n = jnp.maximum(hc * (1.0 / jnp.sqrt(v + EPS)) * gn2_ref[...] + bbn2_ref[...], 0.0)
    xn = jnp.dot(h2n, Wn2b_ref[...], preferred_element_type=jnp.float32) + bn2b_ref[...]

    gid = lax.broadcasted_iota(jnp.int32, (N, G), 1)
    oh = jnp.where(batch_ref[...] == gid, 1.0, 0.0)
    gs = lax.dot_general(oh, xn, (((0,), (0,)), ((), ())),
                         preferred_element_type=jnp.float32,
                         precision=lax.Precision.HIGHEST)
    gc = lax.dot_general(oh, jnp.ones((N, 1), jnp.float32), (((0,), (0,)), ((), ())),
                         preferred_element_type=jnp.float32,
                         precision=lax.Precision.HIGHEST)
    gm = gs / jnp.maximum(gc, 1.0)
    z = jnp.dot(gm, Wg1_ref[...], preferred_element_type=jnp.float32) + bg1_ref[...]
    mz = jnp.mean(z, axis=0, keepdims=True)
    zc = z - mz
    vz = jnp.mean(zc * zc, axis=0, keepdims=True)
    zn = jnp.maximum(zc * (1.0 / jnp.sqrt(vz + EPS)) * gg1_ref[...] + bbg1_ref[...], 0.0)
    un_ref[...] = jnp.dot(zn, Wg2_ref[...], preferred_element_type=jnp.float32) + bg2_ref[...]


def kernel(x, edge_index, edge_attr, u, batch,
           g0, b0, We1, be1, ge1, bbe1, We2, be2,
           Wn1a, bn1a, gn1, bbn1, Wn1b, bn1b,
           Wn2a, bn2a, gn2, bbn2, Wn2b, bn2b,
           Wg1, bg1, gg1, bbg1, Wg2, bg2):
    f32 = jnp.float32
    row = edge_index[0].astype(jnp.int32)
    col = edge_index[1].astype(jnp.int32)

    r1 = lambda a: a.reshape(1, -1)

    xbn, TR, TB = pl.pallas_call(
        _k1_body,
        out_shape=[
            jax.ShapeDtypeStruct((N, D), f32),
            jax.ShapeDtypeStruct((N, W2), f32),
            jax.ShapeDtypeStruct((N, W2), f32),
        ],
    )(x, r1(g0), r1(b0), We1, Wn1a)

    mesh = plsc.VectorSubcoreMesh(core_axis_name="c", subcore_axis_name="s")

    row3 = row.reshape(NW, NCHUNK, K)
    col3 = col.reshape(NW, NCHUNK, K)
    k2 = functools.partial(
        pl.kernel,
        mesh=mesh,
        out_type=[
            jax.ShapeDtypeStruct((E, W2), f32),
            jax.ShapeDtypeStruct((NW, W2), f32),
        ],
        scratch_types=[
            pltpu.VMEM((NCHUNK, K), jnp.int32),
            pltpu.VMEM((NCHUNK, K), jnp.int32),
            pltpu.VMEM((K, W2), f32),
            pltpu.VMEM((K, W2), f32),
            pltpu.VMEM((K, W2), f32),
            pltpu.VMEM((K, W2), f32),
            pltpu.VMEM((W2,), f32),
            pltpu.SemaphoreType.DMA,
            pltpu.SemaphoreType.DMA,
            pltpu.SemaphoreType.DMA,
            pltpu.SemaphoreType.DMA,
            pltpu.SemaphoreType.DMA,
            pltpu.SemaphoreType.DMA,
        ],
    )(_k2_body)
    t, stat1 = k2(TR, TB, row3, col3)

    grid = E // TE
    h3p = pl.pallas_call(
        _k3_body,
        grid=(2, grid),
        in_specs=[
            pl.BlockSpec((TE, W2), lambda p, i: (i, 0)),
            pl.BlockSpec((NW, W2), lambda p, i: (0, 0)),
            pl.BlockSpec((H, H), lambda p, i: (0, 0)),
            pl.BlockSpec((1, H), lambda p, i: (0, 0)),
            pl.BlockSpec((H, H), lambda p, i: (0, 0)),
            pl.BlockSpec((1, H), lambda p, i: (0, 0)),
            pl.BlockSpec((1, H), lambda p, i: (0, 0)),
            pl.BlockSpec((1, H), lambda p, i: (0, 0)),
            pl.BlockSpec((1, H), lambda p, i: (0, 0)),
            pl.BlockSpec((1, H), lambda p, i: (0, 0)),
            pl.BlockSpec((H, H), lambda p, i: (0, 0)),
            pl.BlockSpec((1, H), lambda p, i: (0, 0)),
        ],
        out_specs=pl.BlockSpec((TE, W2), lambda p, i: (i * p, 0)),
        out_shape=jax.ShapeDtypeStruct((E, W2), f32),
        scratch_shapes=[pltpu.VMEM((1, W2), f32)],
    )(t, stat1, We2, r1(be2), Wn1a[D:], r1(bn1a),
      r1(ge1), r1(bbe1), r1(gn1), r1(bbn1), Wn1b, r1(bn1b))

    k4 = functools.partial(
        pl.kernel,
        mesh=mesh,
        out_type=jax.ShapeDtypeStruct((NC, NP, W2), f32),
        scratch_types=[
            pltpu.VMEM((NCHUNK, K), jnp.int32),
            pltpu.VMEM((K, W2), f32),
            pltpu.VMEM((K, W2), f32),
            pltpu.VMEM((ZR, W2), f32),
            pltpu.VMEM_SHARED((NP, W2), f32),
            pltpu.SemaphoreType.DMA,
            pltpu.SemaphoreType.DMA,
            pltpu.SemaphoreType.DMA,
            pltpu.SemaphoreType.DMA,
        ],
    )(_k4_body)
    acc = k4(h3p, col3)

    un = pl.pallas_call(
        _k5_body,
        out_shape=jax.ShapeDtypeStruct((G, OUT), f32),
    )(xbn, acc[0], acc[1], batch.astype(jnp.int32).reshape(N, 1),
      Wn2a[:D], Wn2a[D:], r1(bn2a), r1(gn2), r1(bbn2),
      Wn2b, r1(bn2b), Wg1, r1(bg1), r1(gg1), r1(bbg1), Wg2, r1(bg2))
    return un


# speedup vs baseline: 1.0361x; 1.0361x over previous
import functools

import jax
import jax.numpy as jnp
from jax import lax
from jax.experimental import pallas as pl
from jax.experimental.pallas import tpu as pltpu
from jax.experimental.pallas import tpu_sc as plsc

N = 10000
E = 320000
D = 128
H = 64
OUT = 64
G = 64
EPS = 1e-5
W2 = 2 * H

NC = 2
NS = 16
NW = NC * NS
EW = E // NW
K = 80
NCHUNK = EW // K
TE = 3200
NP = 10240
NROWP = NP // NS
ZR = 64


def _k1_body(x_ref, g0_ref, b0_ref, We1_ref, Wn1a_ref,
             xbn_ref, TR_ref, TB_ref):
    xv = x_ref[...]
    m = jnp.mean(xv, axis=0, keepdims=True)
    xc = xv - m
    v = jnp.mean(xc * xc, axis=0, keepdims=True)
    xbn = xc * (1.0 / jnp.sqrt(v + EPS)) * g0_ref[...] + b0_ref[...]
    xbn_ref[...] = xbn
    A = jnp.dot(xbn, We1_ref[:D, :], preferred_element_type=jnp.float32)
    B = jnp.dot(xbn, We1_ref[D:, :], preferred_element_type=jnp.float32)
    C = jnp.dot(xbn, Wn1a_ref[:D, :], preferred_element_type=jnp.float32)
    TR_ref[...] = jnp.concatenate([A, C], axis=1)
    TB_ref[...] = jnp.concatenate([B, jnp.zeros_like(B)], axis=1)


def _k2_body(TR_hbm, TB_hbm, row3_hbm, col3_hbm,
             t_out, stat_out,
             idxR, idxC, bufR0, bufB0, bufR1, bufB1, stats,
             semR0, semB0, semR1, semB1, semW0, semW1):
    wid = lax.axis_index("s") * NC + lax.axis_index("c")
    base_e = wid * EW

    pltpu.sync_copy(row3_hbm.at[wid], idxR)
    pltpu.sync_copy(col3_hbm.at[wid], idxC)

    def fire(ci, bufR, bufB, semR, semB):
        pltpu.async_copy(TR_hbm.at[idxR.at[ci]], bufR, semR)
        pltpu.async_copy(TB_hbm.at[idxC.at[ci]], bufB, semB)

    def drain_w(bufR, semW):
        pltpu.make_async_copy(bufR, t_out.at[pl.ds(base_e, K)], semW).wait()

    def process(ci, bufR, bufB, semR, semB, semW, carry):
        pltpu.make_async_copy(TR_hbm.at[idxR.at[0]], bufR, semR).wait()
        pltpu.make_async_copy(TB_hbm.at[idxC.at[0]], bufB, semB).wait()

        def row_body(r, c):
            s0, s1, s2, s3, q0, q1, q2, q3 = c
            a0 = bufR[r, pl.ds(0, 16)]
            t0 = a0 + bufB[r, pl.ds(0, 16)]
            bufR[r, pl.ds(0, 16)] = t0
            a1 = bufR[r, pl.ds(16, 16)]
            t1 = a1 + bufB[r, pl.ds(16, 16)]
            bufR[r, pl.ds(16, 16)] = t1
            a2 = bufR[r, pl.ds(32, 16)]
            t2 = a2 + bufB[r, pl.ds(32, 16)]
            bufR[r, pl.ds(32, 16)] = t2
            a3 = bufR[r, pl.ds(48, 16)]
            t3 = a3 + bufB[r, pl.ds(48, 16)]
            bufR[r, pl.ds(48, 16)] = t3
            return (s0 + t0, s1 + t1, s2 + t2, s3 + t3,
                    q0 + t0 * t0, q1 + t1 * t1, q2 + t2 * t2, q3 + t3 * t3)

        carry = lax.fori_loop(0, K, row_body, carry)
        pltpu.async_copy(bufR, t_out.at[pl.ds(base_e + ci * K, K)], semW)
        return carry

    fire(0, bufR0, bufB0, semR0, semB0)
    z16 = jnp.zeros((16,), jnp.float32)
    carry0 = (z16,) * 8

    def pair(pi, carry):
        ci0 = 2 * pi

        @pl.when(pi > 0)
        def _():
            drain_w(bufR1, semW1)

        fire(ci0 + 1, bufR1, bufB1, semR1, semB1)
        carry = process(ci0, bufR0, bufB0, semR0, semB0, semW0, carry)
        drain_w(bufR0, semW0)
        fire(ci0 + 2, bufR0, bufB0, semR0, semB0)
        carry = process(ci0 + 1, bufR1, bufB1, semR1, semB1, semW1, carry)
        return carry

    carry = lax.fori_loop(0, (NCHUNK - 1) // 2, pair, carry0)
    carry = process(NCHUNK - 1, bufR0, bufB0, semR0, semB0, semW0, carry)
    drain_w(bufR0, semW0)
    drain_w(bufR1, semW1)
    s0, s1, s2, s3, q0, q1, q2, q3 = carry
    stats[pl.ds(0, 16)] = s0
    stats[pl.ds(16, 16)] = s1
    stats[pl.ds(32, 16)] = s2
    stats[pl.ds(48, 16)] = s3
    stats[pl.ds(64, 16)] = q0
    stats[pl.ds(80, 16)] = q1
    stats[pl.ds(96, 16)] = q2
    stats[pl.ds(112, 16)] = q3
    pltpu.sync_copy(stats, stat_out.at[wid])


def _k3_body(t_ref, stat_ref, We2_ref, be2_ref, Wn1ab_ref, bn1a_ref, ge1_ref, bbe1_ref,
             gn1_ref, bbn1_ref, hp_ref, aff2_ref, acc_ref):
    i = pl.program_id(0)
    nsteps = pl.num_programs(0)

    @pl.when(i == 0)
    def _():
        acc_ref[...] = jnp.zeros_like(acc_ref)

    ssum = jnp.sum(stat_ref[...], axis=0, keepdims=True)
    m1 = ssum[:, :H] / E
    v1 = ssum[:, H:] / E - m1 * m1
    s1 = ge1_ref[...] * (1.0 / jnp.sqrt(v1 + EPS))
    sh1 = bbe1_ref[...] - m1 * s1

    blk = t_ref[...]
    act = jnp.maximum(blk[:, :H] * s1 + sh1, 0.0)
    e2 = jnp.dot(act, We2_ref[...], preferred_element_type=jnp.float32) + be2_ref[...]
    hin = (jnp.dot(e2, Wn1ab_ref[...], preferred_element_type=jnp.float32)
           + bn1a_ref[...] + blk[:, H:])
    lane = lax.broadcasted_iota(jnp.int32, (TE, H), 1)
    cl = jnp.where(lane == 0, 1.0, 0.0)
    hp_ref[...] = jnp.concatenate([hin, cl], axis=1)
    ps = jnp.sum(hin, axis=0, keepdims=True)
    ps2 = jnp.sum(hin * hin, axis=0, keepdims=True)
    acc_ref[...] += jnp.concatenate([ps, ps2], axis=1)

    @pl.when(i == nsteps - 1)
    def _():
        tot = acc_ref[...]
        m2 = tot[:, :H] / E
        v2 = tot[:, H:] / E - m2 * m2
        s2 = gn1_ref[...] * (1.0 / jnp.sqrt(v2 + EPS))
        sh2 = bbn1_ref[...] - m2 * s2
        aff2_ref[...] = jnp.concatenate([s2, sh2], axis=1)


def _k3b_body(hp_ref, aff2_ref, Wn1b_ref, bn1b_ref, h3p_ref):
    aff = aff2_ref[...]
    s2 = aff[:, :H]
    sh2 = aff[:, H:]
    blk = hp_ref[...]
    r = jnp.maximum(blk[:, :H] * s2 + sh2, 0.0)
    h3 = jnp.dot(r, Wn1b_ref[...], preferred_element_type=jnp.float32) + bn1b_ref[...]
    h3p_ref[...] = jnp.concatenate([h3, blk[:, H:]], axis=1)


def _k4_body(hp_hbm, col3_hbm,
             acc_out,
             idxC, hbuf0, hbuf1, zbuf,
             acc_sp,
             semH0, semH1, semS0, semS1):
    sid = lax.axis_index("s")
    cid = lax.axis_index("c")
    wid = sid * NC + cid
    base_e = wid * EW

    zero16 = jnp.zeros((16,), jnp.float32)

    def zb_body(r, _):
        for j in range(W2 // 16):
            zbuf[r, pl.ds(j * 16, 16)] = zero16
        return 0

    lax.fori_loop(0, ZR, zb_body, 0)

    for k in range(NROWP // ZR):
        r0 = sid * NROWP + k * ZR
        pltpu.sync_copy(zbuf, acc_sp.at[pl.ds(r0, ZR)])
    plsc.subcore_barrier()

    pltpu.sync_copy(col3_hbm.at[wid], idxC)

    def fire_read(ci, hbuf, semH):
        pltpu.async_copy(hp_hbm.at[pl.ds(base_e + ci * K, K)], hbuf, semH)

    def drain_s(hbuf, semS):
        pltpu.make_async_copy(hbuf, acc_sp.at[idxC.at[0]], semS).wait()

    def process(ci, hbuf, semH, semS):
        pltpu.make_async_copy(hp_hbm.at[pl.ds(base_e, K)], hbuf, semH).wait()
        pltpu.async_copy(hbuf, acc_sp.at[idxC.at[ci]], semS, add=True)

    fire_read(0, hbuf0, semH0)

    def pair(pi, _):
        ci0 = 2 * pi

        @pl.when(pi > 0)
        def _():
            drain_s(hbuf1, semS1)

        fire_read(ci0 + 1, hbuf1, semH1)
        process(ci0, hbuf0, semH0, semS0)
        drain_s(hbuf0, semS0)
        fire_read(ci0 + 2, hbuf0, semH0)
        process(ci0 + 1, hbuf1, semH1, semS1)
        return 0

    lax.fori_loop(0, (NCHUNK - 1) // 2, pair, 0)
    process(NCHUNK - 1, hbuf0, semH0, semS0)
    drain_s(hbuf0, semS0)
    drain_s(hbuf1, semS1)
    plsc.subcore_barrier()

    for k in range(NROWP // ZR):
        r0 = sid * NROWP + k * ZR
        pltpu.sync_copy(acc_sp.at[pl.ds(r0, ZR)], zbuf)
        pltpu.sync_copy(zbuf, acc_out.at[cid, pl.ds(r0, ZR)])


def _k5_body(xbn_ref, acc0_ref, acc1_ref, batch_ref,
             Wn2a_t_ref, Wn2a_b_ref, bn2a_ref,
             gn2_ref, bbn2_ref, Wn2b_ref, bn2b_ref,
             Wg1_ref, bg1_ref, gg1_ref, bbg1_ref, Wg2_ref, bg2_ref,
             un_ref):
    tot = acc0_ref[...] + acc1_ref[...]
    acc = tot[:N, :H]
    cnt = jnp.sum(tot[:N, H:], axis=1, keepdims=True)
    agg = acc / jnp.maximum(cnt, 1.0)
    h2 = (jnp.dot(xbn_ref[...], Wn2a_t_ref[...], preferred_element_type=jnp.float32)
          + jnp.dot(agg, Wn2a_b_ref[...], preferred_element_type=jnp.float32)
          + bn2a_ref[...])
    m = jnp.mean(h2, axis=0, keepdims=True)
    hc = h2 - m
    v = jnp.mean(hc * hc, axis=0, keepdims=True)
    h2n = jnp.maximum(hc * (1.0 / jnp.sqrt(v + EPS)) * gn2_ref[...] + bbn2_ref[...], 0.0)
    xn = jnp.dot(h2n, Wn2b_ref[...], preferred_element_type=jnp.float32) + bn2b_ref[...]

    gid = lax.broadcasted_iota(jnp.int32, (N, G), 1)
    oh = jnp.where(batch_ref[...] == gid, 1.0, 0.0)
    gs = lax.dot_general(oh, xn, (((0,), (0,)), ((), ())),
                         preferred_element_type=jnp.float32,
                         precision=lax.Precision.HIGHEST)
    gc = lax.dot_general(oh, jnp.ones((N, 1), jnp.float32), (((0,), (0,)), ((), ())),
                         preferred_element_type=jnp.float32,
                         precision=lax.Precision.HIGHEST)
    gm = gs / jnp.maximum(gc, 1.0)
    z = jnp.dot(gm, Wg1_ref[...], preferred_element_type=jnp.float32) + bg1_ref[...]
    mz = jnp.mean(z, axis=0, keepdims=True)
    zc = z - mz
    vz = jnp.mean(zc * zc, axis=0, keepdims=True)
    zn = jnp.maximum(zc * (1.0 / jnp.sqrt(vz + EPS)) * gg1_ref[...] + bbg1_ref[...], 0.0)
    un_ref[...] = jnp.dot(zn, Wg2_ref[...], preferred_element_type=jnp.float32) + bg2_ref[...]


def kernel(x, edge_index, edge_attr, u, batch,
           g0, b0, We1, be1, ge1, bbe1, We2, be2,
           Wn1a, bn1a, gn1, bbn1, Wn1b, bn1b,
           Wn2a, bn2a, gn2, bbn2, Wn2b, bn2b,
           Wg1, bg1, gg1, bbg1, Wg2, bg2):
    f32 = jnp.float32
    row = edge_index[0].astype(jnp.int32)
    col = edge_index[1].astype(jnp.int32)

    r1 = lambda a: a.reshape(1, -1)

    xbn, TR, TB = pl.pallas_call(
        _k1_body,
        out_shape=[
            jax.ShapeDtypeStruct((N, D), f32),
            jax.ShapeDtypeStruct((N, W2), f32),
            jax.ShapeDtypeStruct((N, W2), f32),
        ],
    )(x, r1(g0), r1(b0), We1, Wn1a)

    mesh = plsc.VectorSubcoreMesh(core_axis_name="c", subcore_axis_name="s")

    row3 = row.reshape(NW, NCHUNK, K)
    col3 = col.reshape(NW, NCHUNK, K)
    k2 = functools.partial(
        pl.kernel,
        mesh=mesh,
        out_type=[
            jax.ShapeDtypeStruct((E, W2), f32),
            jax.ShapeDtypeStruct((NW, W2), f32),
        ],
        scratch_types=[
            pltpu.VMEM((NCHUNK, K), jnp.int32),
            pltpu.VMEM((NCHUNK, K), jnp.int32),
            pltpu.VMEM((K, W2), f32),
            pltpu.VMEM((K, W2), f32),
            pltpu.VMEM((K, W2), f32),
            pltpu.VMEM((K, W2), f32),
            pltpu.VMEM((W2,), f32),
            pltpu.SemaphoreType.DMA,
            pltpu.SemaphoreType.DMA,
            pltpu.SemaphoreType.DMA,
            pltpu.SemaphoreType.DMA,
            pltpu.SemaphoreType.DMA,
            pltpu.SemaphoreType.DMA,
        ],
    )(_k2_body)
    t, stat1 = k2(TR, TB, row3, col3)

    grid = E // TE
    hp, aff2 = pl.pallas_call(
        _k3_body,
        grid=(grid,),
        in_specs=[
            pl.BlockSpec((TE, W2), lambda i: (i, 0)),
            pl.BlockSpec((NW, W2), lambda i: (0, 0)),
            pl.BlockSpec((H, H), lambda i: (0, 0)),
            pl.BlockSpec((1, H), lambda i: (0, 0)),
            pl.BlockSpec((H, H), lambda i: (0, 0)),
            pl.BlockSpec((1, H), lambda i: (0, 0)),
            pl.BlockSpec((1, H), lambda i: (0, 0)),
            pl.BlockSpec((1, H), lambda i: (0, 0)),
            pl.BlockSpec((1, H), lambda i: (0, 0)),
            pl.BlockSpec((1, H), lambda i: (0, 0)),
        ],
        out_specs=[
            pl.BlockSpec((TE, W2), lambda i: (i, 0)),
            pl.BlockSpec((1, W2), lambda i: (0, 0)),
        ],
        out_shape=[
            jax.ShapeDtypeStruct((E, W2), f32),
            jax.ShapeDtypeStruct((1, W2), f32),
        ],
        scratch_shapes=[pltpu.VMEM((1, W2), f32)],
    )(t, stat1, We2, r1(be2), Wn1a[D:], r1(bn1a),
      r1(ge1), r1(bbe1), r1(gn1), r1(bbn1))

    h3p = pl.pallas_call(
        _k3b_body,
        grid=(grid,),
        in_specs=[
            pl.BlockSpec((TE, W2), lambda i: (i, 0)),
            pl.BlockSpec((1, W2), lambda i: (0, 0)),
            pl.BlockSpec((H, H), lambda i: (0, 0)),
            pl.BlockSpec((1, H), lambda i: (0, 0)),
        ],
        out_specs=pl.BlockSpec((TE, W2), lambda i: (i, 0)),
        out_shape=jax.ShapeDtypeStruct((E, W2), f32),
    )(hp, aff2, Wn1b, r1(bn1b))

    k4 = functools.partial(
        pl.kernel,
        mesh=mesh,
        out_type=jax.ShapeDtypeStruct((NC, NP, W2), f32),
        scratch_types=[
            pltpu.VMEM((NCHUNK, K), jnp.int32),
            pltpu.VMEM((K, W2), f32),
            pltpu.VMEM((K, W2), f32),
            pltpu.VMEM((ZR, W2), f32),
            pltpu.VMEM_SHARED((NP, W2), f32),
            pltpu.SemaphoreType.DMA,
            pltpu.SemaphoreType.DMA,
            pltpu.SemaphoreType.DMA,
            pltpu.SemaphoreType.DMA,
        ],
    )(_k4_body)
    acc = k4(h3p, col3)

    un = pl.pallas_call(
        _k5_body,
        out_shape=jax.ShapeDtypeStruct((G, OUT), f32),
    )(xbn, acc[0], acc[1], batch.astype(jnp.int32).reshape(N, 1),
      Wn2a[:D], Wn2a[D:], r1(bn2a), r1(gn2), r1(bbn2),
      Wn2b, r1(bn2b), Wg1, r1(bg1), r1(gg1), r1(bbg1), Wg2, r1(bg2))
    return un


# speedup vs baseline: 1.1572x; 1.1169x over previous
import functools

import jax
import jax.numpy as jnp
from jax import lax
from jax.experimental import pallas as pl
from jax.experimental.pallas import tpu as pltpu
from jax.experimental.pallas import tpu_sc as plsc

N = 10000
E = 320000
D = 128
H = 64
OUT = 64
G = 64
EPS = 1e-5
W2 = 2 * H

NC = 2
NS = 16
NW = NC * NS
EW = E // NW
K = 80
NCHUNK = EW // K
TE = 8000
NP = 10240
NROWP = NP // NS
ZR = 64


def _k1_body(x_ref, g0_ref, b0_ref, We1_ref, Wn1a_ref,
             xbn_ref, TR_ref, TB_ref):
    xv = x_ref[...]
    m = jnp.mean(xv, axis=0, keepdims=True)
    xc = xv - m
    v = jnp.mean(xc * xc, axis=0, keepdims=True)
    xbn = xc * (1.0 / jnp.sqrt(v + EPS)) * g0_ref[...] + b0_ref[...]
    xbn_ref[...] = xbn
    A = jnp.dot(xbn, We1_ref[:D, :], preferred_element_type=jnp.float32)
    B = jnp.dot(xbn, We1_ref[D:, :], preferred_element_type=jnp.float32)
    C = jnp.dot(xbn, Wn1a_ref[:D, :], preferred_element_type=jnp.float32)
    TR_ref[...] = jnp.concatenate([A, C], axis=1)
    TB_ref[...] = jnp.concatenate([B, jnp.zeros_like(B)], axis=1)


def _k2_body(TR_hbm, TB_hbm, row3_hbm, col3_hbm,
             t_out, stat_out,
             idxR, idxC, bufR0, bufB0, bufR1, bufB1, stats,
             semR0, semB0, semR1, semB1, semW0, semW1):
    wid = lax.axis_index("s") * NC + lax.axis_index("c")
    base_e = wid * EW

    pltpu.sync_copy(row3_hbm.at[wid], idxR)
    pltpu.sync_copy(col3_hbm.at[wid], idxC)

    def fire(ci, bufR, bufB, semR, semB):
        pltpu.async_copy(TR_hbm.at[idxR.at[ci]], bufR, semR)
        pltpu.async_copy(TB_hbm.at[idxC.at[ci]], bufB, semB)

    def drain_w(bufR, semW):
        pltpu.make_async_copy(bufR, t_out.at[pl.ds(base_e, K)], semW).wait()

    def process(ci, bufR, bufB, semR, semB, semW, carry):
        pltpu.make_async_copy(TR_hbm.at[idxR.at[0]], bufR, semR).wait()
        pltpu.make_async_copy(TB_hbm.at[idxC.at[0]], bufB, semB).wait()

        def row_body(r, c):
            s0, s1, s2, s3, q0, q1, q2, q3 = c
            a0 = bufR[r, pl.ds(0, 16)]
            t0 = a0 + bufB[r, pl.ds(0, 16)]
            bufR[r, pl.ds(0, 16)] = t0
            a1 = bufR[r, pl.ds(16, 16)]
            t1 = a1 + bufB[r, pl.ds(16, 16)]
            bufR[r, pl.ds(16, 16)] = t1
            a2 = bufR[r, pl.ds(32, 16)]
            t2 = a2 + bufB[r, pl.ds(32, 16)]
            bufR[r, pl.ds(32, 16)] = t2
            a3 = bufR[r, pl.ds(48, 16)]
            t3 = a3 + bufB[r, pl.ds(48, 16)]
            bufR[r, pl.ds(48, 16)] = t3
            return (s0 + t0, s1 + t1, s2 + t2, s3 + t3,
                    q0 + t0 * t0, q1 + t1 * t1, q2 + t2 * t2, q3 + t3 * t3)

        carry = lax.fori_loop(0, K, row_body, carry)
        pltpu.async_copy(bufR, t_out.at[pl.ds(base_e + ci * K, K)], semW)
        return carry

    fire(0, bufR0, bufB0, semR0, semB0)
    z16 = jnp.zeros((16,), jnp.float32)
    carry0 = (z16,) * 8

    def pair(pi, carry):
        ci0 = 2 * pi

        @pl.when(pi > 0)
        def _():
            drain_w(bufR1, semW1)

        fire(ci0 + 1, bufR1, bufB1, semR1, semB1)
        carry = process(ci0, bufR0, bufB0, semR0, semB0, semW0, carry)
        drain_w(bufR0, semW0)
        fire(ci0 + 2, bufR0, bufB0, semR0, semB0)
        carry = process(ci0 + 1, bufR1, bufB1, semR1, semB1, semW1, carry)
        return carry

    carry = lax.fori_loop(0, (NCHUNK - 1) // 2, pair, carry0)
    carry = process(NCHUNK - 1, bufR0, bufB0, semR0, semB0, semW0, carry)
    drain_w(bufR0, semW0)
    drain_w(bufR1, semW1)
    s0, s1, s2, s3, q0, q1, q2, q3 = carry
    stats[pl.ds(0, 16)] = s0
    stats[pl.ds(16, 16)] = s1
    stats[pl.ds(32, 16)] = s2
    stats[pl.ds(48, 16)] = s3
    stats[pl.ds(64, 16)] = q0
    stats[pl.ds(80, 16)] = q1
    stats[pl.ds(96, 16)] = q2
    stats[pl.ds(112, 16)] = q3
    pltpu.sync_copy(stats, stat_out.at[wid])


def _k3_body(t_ref, stat_ref, We2_ref, be2_ref, Wn1ab_ref, bn1a_ref, ge1_ref, bbe1_ref,
             gn1_ref, bbn1_ref, hp_ref, aff2_ref, acc_ref):
    i = pl.program_id(0)
    nsteps = pl.num_programs(0)

    @pl.when(i == 0)
    def _():
        acc_ref[...] = jnp.zeros_like(acc_ref)

    ssum = jnp.sum(stat_ref[...], axis=0, keepdims=True)
    m1 = ssum[:, :H] / E
    v1 = ssum[:, H:] / E - m1 * m1
    s1 = ge1_ref[...] * (1.0 / jnp.sqrt(v1 + EPS))
    sh1 = bbe1_ref[...] - m1 * s1

    blk = t_ref[...]
    act = jnp.maximum(blk[:, :H] * s1 + sh1, 0.0)
    e2 = jnp.dot(act, We2_ref[...], preferred_element_type=jnp.float32) + be2_ref[...]
    hin = (jnp.dot(e2, Wn1ab_ref[...], preferred_element_type=jnp.float32)
           + bn1a_ref[...] + blk[:, H:])
    lane = lax.broadcasted_iota(jnp.int32, (TE, H), 1)
    cl = jnp.where(lane == 0, 1.0, 0.0)
    hp_ref[...] = jnp.concatenate([hin, cl], axis=1)
    ps = jnp.sum(hin, axis=0, keepdims=True)
    ps2 = jnp.sum(hin * hin, axis=0, keepdims=True)
    acc_ref[...] += jnp.concatenate([ps, ps2], axis=1)

    @pl.when(i == nsteps - 1)
    def _():
        tot = acc_ref[...]
        m2 = tot[:, :H] / E
        v2 = tot[:, H:] / E - m2 * m2
        s2 = gn1_ref[...] * (1.0 / jnp.sqrt(v2 + EPS))
        sh2 = bbn1_ref[...] - m2 * s2
        aff2_ref[...] = jnp.concatenate([s2, sh2], axis=1)


def _k3b_body(hp_ref, aff2_ref, Wn1b_ref, bn1b_ref, h3p_ref):
    aff = aff2_ref[...]
    s2 = aff[:, :H]
    sh2 = aff[:, H:]
    blk = hp_ref[...]
    r = jnp.maximum(blk[:, :H] * s2 + sh2, 0.0)
    h3 = jnp.dot(r, Wn1b_ref[...], preferred_element_type=jnp.float32) + bn1b_ref[...]
    h3p_ref[...] = jnp.concatenate([h3, blk[:, H:]], axis=1)


def _k4_body(hp_hbm, col3_hbm,
             acc_out,
             idxC, hbuf0, hbuf1, zbuf,
             acc_sp,
             semH0, semH1, semS0, semS1):
    sid = lax.axis_index("s")
    cid = lax.axis_index("c")
    wid = sid * NC + cid
    base_e = wid * EW

    zero16 = jnp.zeros((16,), jnp.float32)

    def zb_body(r, _):
        for j in range(W2 // 16):
            zbuf[r, pl.ds(j * 16, 16)] = zero16
        return 0

    lax.fori_loop(0, ZR, zb_body, 0)

    for k in range(NROWP // ZR):
        r0 = sid * NROWP + k * ZR
        pltpu.sync_copy(zbuf, acc_sp.at[pl.ds(r0, ZR)])
    plsc.subcore_barrier()

    pltpu.sync_copy(col3_hbm.at[wid], idxC)

    def fire_read(ci, hbuf, semH):
        pltpu.async_copy(hp_hbm.at[pl.ds(base_e + ci * K, K)], hbuf, semH)

    def drain_s(hbuf, semS):
        pltpu.make_async_copy(hbuf, acc_sp.at[idxC.at[0]], semS).wait()

    def process(ci, hbuf, semH, semS):
        pltpu.make_async_copy(hp_hbm.at[pl.ds(base_e, K)], hbuf, semH).wait()
        pltpu.async_copy(hbuf, acc_sp.at[idxC.at[ci]], semS, add=True)

    fire_read(0, hbuf0, semH0)

    def pair(pi, _):
        ci0 = 2 * pi

        @pl.when(pi > 0)
        def _():
            drain_s(hbuf1, semS1)

        fire_read(ci0 + 1, hbuf1, semH1)
        process(ci0, hbuf0, semH0, semS0)
        drain_s(hbuf0, semS0)
        fire_read(ci0 + 2, hbuf0, semH0)
        process(ci0 + 1, hbuf1, semH1, semS1)
        return 0

    lax.fori_loop(0, (NCHUNK - 1) // 2, pair, 0)
    process(NCHUNK - 1, hbuf0, semH0, semS0)
    drain_s(hbuf0, semS0)
    drain_s(hbuf1, semS1)
    plsc.subcore_barrier()

    for k in range(NROWP // ZR):
        r0 = sid * NROWP + k * ZR
        pltpu.sync_copy(acc_sp.at[pl.ds(r0, ZR)], zbuf)
        pltpu.sync_copy(zbuf, acc_out.at[cid, pl.ds(r0, ZR)])


def _k5_body(xbn_ref, acc0_ref, acc1_ref, batch_ref,
             Wn2a_t_ref, Wn2a_b_ref, bn2a_ref,
             gn2_ref, bbn2_ref, Wn2b_ref, bn2b_ref,
             Wg1_ref, bg1_ref, gg1_ref, bbg1_ref, Wg2_ref, bg2_ref,
             un_ref):
    tot = acc0_ref[...] + acc1_ref[...]
    acc = tot[:N, :H]
    cnt = jnp.sum(tot[:N, H:], axis=1, keepdims=True)
    agg = acc / jnp.maximum(cnt, 1.0)
    h2 = (jnp.dot(xbn_ref[...], Wn2a_t_ref[...], preferred_element_type=jnp.float32)
          + jnp.dot(agg, Wn2a_b_ref[...], preferred_element_type=jnp.float32)
          + bn2a_ref[...])
    m = jnp.mean(h2, axis=0, keepdims=True)
    hc = h2 - m
    v = jnp.mean(hc * hc, axis=0, keepdims=True)
    h2n = jnp.maximum(hc * (1.0 / jnp.sqrt(v + EPS)) * gn2_ref[...] + bbn2_ref[...], 0.0)
    xn = jnp.dot(h2n, Wn2b_ref[...], preferred_element_type=jnp.float32) + bn2b_ref[...]

    gid = lax.broadcasted_iota(jnp.int32, (N, G), 1)
    oh = jnp.where(batch_ref[...] == gid, 1.0, 0.0)
    gs = lax.dot_general(oh, xn, (((0,), (0,)), ((), ())),
                         preferred_element_type=jnp.float32,
                         precision=lax.Precision.HIGHEST)
    gc = lax.dot_general(oh, jnp.ones((N, 1), jnp.float32), (((0,), (0,)), ((), ())),
                         preferred_element_type=jnp.float32,
                         precision=lax.Precision.HIGHEST)
    gm = gs / jnp.maximum(gc, 1.0)
    z = jnp.dot(gm, Wg1_ref[...], preferred_element_type=jnp.float32) + bg1_ref[...]
    mz = jnp.mean(z, axis=0, keepdims=True)
    zc = z - mz
    vz = jnp.mean(zc * zc, axis=0, keepdims=True)
    zn = jnp.maximum(zc * (1.0 / jnp.sqrt(vz + EPS)) * gg1_ref[...] + bbg1_ref[...], 0.0)
    un_ref[...] = jnp.dot(zn, Wg2_ref[...], preferred_element_type=jnp.float32) + bg2_ref[...]


def kernel(x, edge_index, edge_attr, u, batch,
           g0, b0, We1, be1, ge1, bbe1, We2, be2,
           Wn1a, bn1a, gn1, bbn1, Wn1b, bn1b,
           Wn2a, bn2a, gn2, bbn2, Wn2b, bn2b,
           Wg1, bg1, gg1, bbg1, Wg2, bg2):
    f32 = jnp.float32
    row = edge_index[0].astype(jnp.int32)
    col = edge_index[1].astype(jnp.int32)

    r1 = lambda a: a.reshape(1, -1)

    xbn, TR, TB = pl.pallas_call(
        _k1_body,
        out_shape=[
            jax.ShapeDtypeStruct((N, D), f32),
            jax.ShapeDtypeStruct((N, W2), f32),
            jax.ShapeDtypeStruct((N, W2), f32),
        ],
    )(x, r1(g0), r1(b0), We1, Wn1a)

    mesh = plsc.VectorSubcoreMesh(core_axis_name="c", subcore_axis_name="s")

    row3 = row.reshape(NW, NCHUNK, K)
    col3 = col.reshape(NW, NCHUNK, K)
    k2 = functools.partial(
        pl.kernel,
        mesh=mesh,
        out_type=[
            jax.ShapeDtypeStruct((E, W2), f32),
            jax.ShapeDtypeStruct((NW, W2), f32),
        ],
        scratch_types=[
            pltpu.VMEM((NCHUNK, K), jnp.int32),
            pltpu.VMEM((NCHUNK, K), jnp.int32),
            pltpu.VMEM((K, W2), f32),
            pltpu.VMEM((K, W2), f32),
            pltpu.VMEM((K, W2), f32),
            pltpu.VMEM((K, W2), f32),
            pltpu.VMEM((W2,), f32),
            pltpu.SemaphoreType.DMA,
            pltpu.SemaphoreType.DMA,
            pltpu.SemaphoreType.DMA,
            pltpu.SemaphoreType.DMA,
            pltpu.SemaphoreType.DMA,
            pltpu.SemaphoreType.DMA,
        ],
    )(_k2_body)
    t, stat1 = k2(TR, TB, row3, col3)

    grid = E // TE
    hp, aff2 = pl.pallas_call(
        _k3_body,
        grid=(grid,),
        in_specs=[
            pl.BlockSpec((TE, W2), lambda i: (i, 0)),
            pl.BlockSpec((NW, W2), lambda i: (0, 0)),
            pl.BlockSpec((H, H), lambda i: (0, 0)),
            pl.BlockSpec((1, H), lambda i: (0, 0)),
            pl.BlockSpec((H, H), lambda i: (0, 0)),
            pl.BlockSpec((1, H), lambda i: (0, 0)),
            pl.BlockSpec((1, H), lambda i: (0, 0)),
            pl.BlockSpec((1, H), lambda i: (0, 0)),
            pl.BlockSpec((1, H), lambda i: (0, 0)),
            pl.BlockSpec((1, H), lambda i: (0, 0)),
        ],
        out_specs=[
            pl.BlockSpec((TE, W2), lambda i: (i, 0)),
            pl.BlockSpec((1, W2), lambda i: (0, 0)),
        ],
        out_shape=[
            jax.ShapeDtypeStruct((E, W2), f32),
            jax.ShapeDtypeStruct((1, W2), f32),
        ],
        scratch_shapes=[pltpu.VMEM((1, W2), f32)],
    )(t, stat1, We2, r1(be2), Wn1a[D:], r1(bn1a),
      r1(ge1), r1(bbe1), r1(gn1), r1(bbn1))

    h3p = pl.pallas_call(
        _k3b_body,
        grid=(grid,),
        in_specs=[
            pl.BlockSpec((TE, W2), lambda i: (i, 0)),
            pl.BlockSpec((1, W2), lambda i: (0, 0)),
            pl.BlockSpec((H, H), lambda i: (0, 0)),
            pl.BlockSpec((1, H), lambda i: (0, 0)),
        ],
        out_specs=pl.BlockSpec((TE, W2), lambda i: (i, 0)),
        out_shape=jax.ShapeDtypeStruct((E, W2), f32),
    )(hp, aff2, Wn1b, r1(bn1b))

    k4 = functools.partial(
        pl.kernel,
        mesh=mesh,
        out_type=jax.ShapeDtypeStruct((NC, NP, W2), f32),
        scratch_types=[
            pltpu.VMEM((NCHUNK, K), jnp.int32),
            pltpu.VMEM((K, W2), f32),
            pltpu.VMEM((K, W2), f32),
            pltpu.VMEM((ZR, W2), f32),
            pltpu.VMEM_SHARED((NP, W2), f32),
            pltpu.SemaphoreType.DMA,
            pltpu.SemaphoreType.DMA,
            pltpu.SemaphoreType.DMA,
            pltpu.SemaphoreType.DMA,
        ],
    )(_k4_body)
    acc = k4(h3p, col3)

    un = pl.pallas_call(
        _k5_body,
        out_shape=jax.ShapeDtypeStruct((G, OUT), f32),
    )(xbn, acc[0], acc[1], batch.astype(jnp.int32).reshape(N, 1),
      Wn2a[:D], Wn2a[D:], r1(bn2a), r1(gn2), r1(bbn2),
      Wn2b, r1(bn2b), Wg1, r1(bg1), r1(gg1), r1(bbg1), Wg2, r1(bg2))
    return un


# speedup vs baseline: 1.1813x; 1.0208x over previous
import functools

import jax
import jax.numpy as jnp
from jax import lax
from jax.experimental import pallas as pl
from jax.experimental.pallas import tpu as pltpu
from jax.experimental.pallas import tpu_sc as plsc

N = 10000
E = 320000
D = 128
H = 64
OUT = 64
G = 64
EPS = 1e-5
W2 = 2 * H

NC = 2
NS = 16
NW = NC * NS
EW = E // NW
K = 80
NCHUNK = EW // K
TE = 16000
NP = 10240
NROWP = NP // NS
ZR = 64


def _k1_body(x_ref, g0_ref, b0_ref, We1_ref, Wn1a_ref,
             xbn_ref, TR_ref, TB_ref):
    xv = x_ref[...]
    m = jnp.mean(xv, axis=0, keepdims=True)
    xc = xv - m
    v = jnp.mean(xc * xc, axis=0, keepdims=True)
    xbn = xc * (1.0 / jnp.sqrt(v + EPS)) * g0_ref[...] + b0_ref[...]
    xbn_ref[...] = xbn
    A = jnp.dot(xbn, We1_ref[:D, :], preferred_element_type=jnp.float32)
    B = jnp.dot(xbn, We1_ref[D:, :], preferred_element_type=jnp.float32)
    C = jnp.dot(xbn, Wn1a_ref[:D, :], preferred_element_type=jnp.float32)
    TR_ref[...] = jnp.concatenate([A, C], axis=1)
    TB_ref[...] = jnp.concatenate([B, jnp.zeros_like(B)], axis=1)


def _k2_body(TR_hbm, TB_hbm, row3_hbm, col3_hbm,
             t_out, stat_out,
             idxR, idxC, bufR0, bufB0, bufR1, bufB1, stats,
             semR0, semB0, semR1, semB1, semW0, semW1):
    wid = lax.axis_index("s") * NC + lax.axis_index("c")
    base_e = wid * EW

    pltpu.sync_copy(row3_hbm.at[wid], idxR)
    pltpu.sync_copy(col3_hbm.at[wid], idxC)

    def fire(ci, bufR, bufB, semR, semB):
        pltpu.async_copy(TR_hbm.at[idxR.at[ci]], bufR, semR)
        pltpu.async_copy(TB_hbm.at[idxC.at[ci]], bufB, semB)

    def drain_w(bufR, semW):
        pltpu.make_async_copy(bufR, t_out.at[pl.ds(base_e, K)], semW).wait()

    def process(ci, bufR, bufB, semR, semB, semW, carry):
        pltpu.make_async_copy(TR_hbm.at[idxR.at[0]], bufR, semR).wait()
        pltpu.make_async_copy(TB_hbm.at[idxC.at[0]], bufB, semB).wait()

        def row_body(r, c):
            s0, s1, s2, s3, q0, q1, q2, q3 = c
            a0 = bufR[r, pl.ds(0, 16)]
            t0 = a0 + bufB[r, pl.ds(0, 16)]
            bufR[r, pl.ds(0, 16)] = t0
            a1 = bufR[r, pl.ds(16, 16)]
            t1 = a1 + bufB[r, pl.ds(16, 16)]
            bufR[r, pl.ds(16, 16)] = t1
            a2 = bufR[r, pl.ds(32, 16)]
            t2 = a2 + bufB[r, pl.ds(32, 16)]
            bufR[r, pl.ds(32, 16)] = t2
            a3 = bufR[r, pl.ds(48, 16)]
            t3 = a3 + bufB[r, pl.ds(48, 16)]
            bufR[r, pl.ds(48, 16)] = t3
            return (s0 + t0, s1 + t1, s2 + t2, s3 + t3,
                    q0 + t0 * t0, q1 + t1 * t1, q2 + t2 * t2, q3 + t3 * t3)

        carry = lax.fori_loop(0, K, row_body, carry)
        pltpu.async_copy(bufR, t_out.at[pl.ds(base_e + ci * K, K)], semW)
        return carry

    fire(0, bufR0, bufB0, semR0, semB0)
    z16 = jnp.zeros((16,), jnp.float32)
    carry0 = (z16,) * 8

    def pair(pi, carry):
        ci0 = 2 * pi

        @pl.when(pi > 0)
        def _():
            drain_w(bufR1, semW1)

        fire(ci0 + 1, bufR1, bufB1, semR1, semB1)
        carry = process(ci0, bufR0, bufB0, semR0, semB0, semW0, carry)
        drain_w(bufR0, semW0)
        fire(ci0 + 2, bufR0, bufB0, semR0, semB0)
        carry = process(ci0 + 1, bufR1, bufB1, semR1, semB1, semW1, carry)
        return carry

    carry = lax.fori_loop(0, (NCHUNK - 1) // 2, pair, carry0)
    carry = process(NCHUNK - 1, bufR0, bufB0, semR0, semB0, semW0, carry)
    drain_w(bufR0, semW0)
    drain_w(bufR1, semW1)
    s0, s1, s2, s3, q0, q1, q2, q3 = carry
    stats[pl.ds(0, 16)] = s0
    stats[pl.ds(16, 16)] = s1
    stats[pl.ds(32, 16)] = s2
    stats[pl.ds(48, 16)] = s3
    stats[pl.ds(64, 16)] = q0
    stats[pl.ds(80, 16)] = q1
    stats[pl.ds(96, 16)] = q2
    stats[pl.ds(112, 16)] = q3
    pltpu.sync_copy(stats, stat_out.at[wid])


def _k3_body(t_ref, stat_ref, We2_ref, be2_ref, Wn1ab_ref, bn1a_ref, ge1_ref, bbe1_ref,
             gn1_ref, bbn1_ref, hp_ref, aff2_ref, acc_ref):
    i = pl.program_id(0)
    nsteps = pl.num_programs(0)

    @pl.when(i == 0)
    def _():
        acc_ref[...] = jnp.zeros_like(acc_ref)

    ssum = jnp.sum(stat_ref[...], axis=0, keepdims=True)
    m1 = ssum[:, :H] / E
    v1 = ssum[:, H:] / E - m1 * m1
    s1 = ge1_ref[...] * (1.0 / jnp.sqrt(v1 + EPS))
    sh1 = bbe1_ref[...] - m1 * s1

    blk = t_ref[...]
    act = jnp.maximum(blk[:, :H] * s1 + sh1, 0.0)
    e2 = jnp.dot(act, We2_ref[...], preferred_element_type=jnp.float32) + be2_ref[...]
    hin = (jnp.dot(e2, Wn1ab_ref[...], preferred_element_type=jnp.float32)
           + bn1a_ref[...] + blk[:, H:])
    lane = lax.broadcasted_iota(jnp.int32, (TE, H), 1)
    cl = jnp.where(lane == 0, 1.0, 0.0)
    hp_ref[...] = jnp.concatenate([hin, cl], axis=1)
    ps = jnp.sum(hin, axis=0, keepdims=True)
    ps2 = jnp.sum(hin * hin, axis=0, keepdims=True)
    acc_ref[...] += jnp.concatenate([ps, ps2], axis=1)

    @pl.when(i == nsteps - 1)
    def _():
        tot = acc_ref[...]
        m2 = tot[:, :H] / E
        v2 = tot[:, H:] / E - m2 * m2
        s2 = gn1_ref[...] * (1.0 / jnp.sqrt(v2 + EPS))
        sh2 = bbn1_ref[...] - m2 * s2
        aff2_ref[...] = jnp.concatenate([s2, sh2], axis=1)


def _k3b_body(hp_ref, aff2_ref, Wn1b_ref, bn1b_ref, h3p_ref):
    aff = aff2_ref[...]
    s2 = aff[:, :H]
    sh2 = aff[:, H:]
    blk = hp_ref[...]
    r = jnp.maximum(blk[:, :H] * s2 + sh2, 0.0)
    h3 = jnp.dot(r, Wn1b_ref[...], preferred_element_type=jnp.float32) + bn1b_ref[...]
    h3p_ref[...] = jnp.concatenate([h3, blk[:, H:]], axis=1)


def _k4_body(hp_hbm, col3_hbm,
             acc_out,
             idxC, hbuf0, hbuf1, zbuf,
             acc_sp,
             semH0, semH1, semS0, semS1):
    sid = lax.axis_index("s")
    cid = lax.axis_index("c")
    wid = sid * NC + cid
    base_e = wid * EW

    zero16 = jnp.zeros((16,), jnp.float32)

    def zb_body(r, _):
        for j in range(W2 // 16):
            zbuf[r, pl.ds(j * 16, 16)] = zero16
        return 0

    lax.fori_loop(0, ZR, zb_body, 0)

    for k in range(NROWP // ZR):
        r0 = sid * NROWP + k * ZR
        pltpu.sync_copy(zbuf, acc_sp.at[pl.ds(r0, ZR)])
    plsc.subcore_barrier()

    pltpu.sync_copy(col3_hbm.at[wid], idxC)

    def fire_read(ci, hbuf, semH):
        pltpu.async_copy(hp_hbm.at[pl.ds(base_e + ci * K, K)], hbuf, semH)

    def drain_s(hbuf, semS):
        pltpu.make_async_copy(hbuf, acc_sp.at[idxC.at[0]], semS).wait()

    def process(ci, hbuf, semH, semS):
        pltpu.make_async_copy(hp_hbm.at[pl.ds(base_e, K)], hbuf, semH).wait()
        pltpu.async_copy(hbuf, acc_sp.at[idxC.at[ci]], semS, add=True)

    fire_read(0, hbuf0, semH0)

    def pair(pi, _):
        ci0 = 2 * pi

        @pl.when(pi > 0)
        def _():
            drain_s(hbuf1, semS1)

        fire_read(ci0 + 1, hbuf1, semH1)
        process(ci0, hbuf0, semH0, semS0)
        drain_s(hbuf0, semS0)
        fire_read(ci0 + 2, hbuf0, semH0)
        process(ci0 + 1, hbuf1, semH1, semS1)
        return 0

    lax.fori_loop(0, (NCHUNK - 1) // 2, pair, 0)
    process(NCHUNK - 1, hbuf0, semH0, semS0)
    drain_s(hbuf0, semS0)
    drain_s(hbuf1, semS1)
    plsc.subcore_barrier()

    for k in range(NROWP // ZR):
        r0 = sid * NROWP + k * ZR
        pltpu.sync_copy(acc_sp.at[pl.ds(r0, ZR)], zbuf)
        pltpu.sync_copy(zbuf, acc_out.at[cid, pl.ds(r0, ZR)])


def _k5_body(xbn_ref, acc0_ref, acc1_ref, batch_ref,
             Wn2a_t_ref, Wn2a_b_ref, bn2a_ref,
             gn2_ref, bbn2_ref, Wn2b_ref, bn2b_ref,
             Wg1_ref, bg1_ref, gg1_ref, bbg1_ref, Wg2_ref, bg2_ref,
             un_ref):
    tot = acc0_ref[...] + acc1_ref[...]
    acc = tot[:N, :H]
    cnt = jnp.sum(tot[:N, H:], axis=1, keepdims=True)
    agg = acc / jnp.maximum(cnt, 1.0)
    h2 = (jnp.dot(xbn_ref[...], Wn2a_t_ref[...], preferred_element_type=jnp.float32)
          + jnp.dot(agg, Wn2a_b_ref[...], preferred_element_type=jnp.float32)
          + bn2a_ref[...])
    m = jnp.mean(h2, axis=0, keepdims=True)
    hc = h2 - m
    v = jnp.mean(hc * hc, axis=0, keepdims=True)
    h2n = jnp.maximum(hc * (1.0 / jnp.sqrt(v + EPS)) * gn2_ref[...] + bbn2_ref[...], 0.0)
    xn = jnp.dot(h2n, Wn2b_ref[...], preferred_element_type=jnp.float32) + bn2b_ref[...]

    gid = lax.broadcasted_iota(jnp.int32, (N, G), 1)
    oh = jnp.where(batch_ref[...] == gid, 1.0, 0.0)
    gs = lax.dot_general(oh, xn, (((0,), (0,)), ((), ())),
                         preferred_element_type=jnp.float32,
                         precision=lax.Precision.HIGHEST)
    gc = lax.dot_general(oh, jnp.ones((N, 1), jnp.float32), (((0,), (0,)), ((), ())),
                         preferred_element_type=jnp.float32,
                         precision=lax.Precision.HIGHEST)
    gm = gs / jnp.maximum(gc, 1.0)
    z = jnp.dot(gm, Wg1_ref[...], preferred_element_type=jnp.float32) + bg1_ref[...]
    mz = jnp.mean(z, axis=0, keepdims=True)
    zc = z - mz
    vz = jnp.mean(zc * zc, axis=0, keepdims=True)
    zn = jnp.maximum(zc * (1.0 / jnp.sqrt(vz + EPS)) * gg1_ref[...] + bbg1_ref[...], 0.0)
    un_ref[...] = jnp.dot(zn, Wg2_ref[...], preferred_element_type=jnp.float32) + bg2_ref[...]


def kernel(x, edge_index, edge_attr, u, batch,
           g0, b0, We1, be1, ge1, bbe1, We2, be2,
           Wn1a, bn1a, gn1, bbn1, Wn1b, bn1b,
           Wn2a, bn2a, gn2, bbn2, Wn2b, bn2b,
           Wg1, bg1, gg1, bbg1, Wg2, bg2):
    f32 = jnp.float32
    row = edge_index[0].astype(jnp.int32)
    col = edge_index[1].astype(jnp.int32)

    r1 = lambda a: a.reshape(1, -1)

    xbn, TR, TB = pl.pallas_call(
        _k1_body,
        out_shape=[
            jax.ShapeDtypeStruct((N, D), f32),
            jax.ShapeDtypeStruct((N, W2), f32),
            jax.ShapeDtypeStruct((N, W2), f32),
        ],
    )(x, r1(g0), r1(b0), We1, Wn1a)

    mesh = plsc.VectorSubcoreMesh(core_axis_name="c", subcore_axis_name="s")

    row3 = row.reshape(NW, NCHUNK, K)
    col3 = col.reshape(NW, NCHUNK, K)
    k2 = functools.partial(
        pl.kernel,
        mesh=mesh,
        out_type=[
            jax.ShapeDtypeStruct((E, W2), f32),
            jax.ShapeDtypeStruct((NW, W2), f32),
        ],
        scratch_types=[
            pltpu.VMEM((NCHUNK, K), jnp.int32),
            pltpu.VMEM((NCHUNK, K), jnp.int32),
            pltpu.VMEM((K, W2), f32),
            pltpu.VMEM((K, W2), f32),
            pltpu.VMEM((K, W2), f32),
            pltpu.VMEM((K, W2), f32),
            pltpu.VMEM((W2,), f32),
            pltpu.SemaphoreType.DMA,
            pltpu.SemaphoreType.DMA,
            pltpu.SemaphoreType.DMA,
            pltpu.SemaphoreType.DMA,
            pltpu.SemaphoreType.DMA,
            pltpu.SemaphoreType.DMA,
        ],
    )(_k2_body)
    t, stat1 = k2(TR, TB, row3, col3)

    grid = E // TE
    hp, aff2 = pl.pallas_call(
        _k3_body,
        grid=(grid,),
        in_specs=[
            pl.BlockSpec((TE, W2), lambda i: (i, 0)),
            pl.BlockSpec((NW, W2), lambda i: (0, 0)),
            pl.BlockSpec((H, H), lambda i: (0, 0)),
            pl.BlockSpec((1, H), lambda i: (0, 0)),
            pl.BlockSpec((H, H), lambda i: (0, 0)),
            pl.BlockSpec((1, H), lambda i: (0, 0)),
            pl.BlockSpec((1, H), lambda i: (0, 0)),
            pl.BlockSpec((1, H), lambda i: (0, 0)),
            pl.BlockSpec((1, H), lambda i: (0, 0)),
            pl.BlockSpec((1, H), lambda i: (0, 0)),
        ],
        out_specs=[
            pl.BlockSpec((TE, W2), lambda i: (i, 0)),
            pl.BlockSpec((1, W2), lambda i: (0, 0)),
        ],
        out_shape=[
            jax.ShapeDtypeStruct((E, W2), f32),
            jax.ShapeDtypeStruct((1, W2), f32),
        ],
        scratch_shapes=[pltpu.VMEM((1, W2), f32)],
    )(t, stat1, We2, r1(be2), Wn1a[D:], r1(bn1a),
      r1(ge1), r1(bbe1), r1(gn1), r1(bbn1))

    h3p = pl.pallas_call(
        _k3b_body,
        grid=(grid,),
        in_specs=[
            pl.BlockSpec((TE, W2), lambda i: (i, 0)),
            pl.BlockSpec((1, W2), lambda i: (0, 0)),
            pl.BlockSpec((H, H), lambda i: (0, 0)),
            pl.BlockSpec((1, H), lambda i: (0, 0)),
        ],
        out_specs=pl.BlockSpec((TE, W2), lambda i: (i, 0)),
        out_shape=jax.ShapeDtypeStruct((E, W2), f32),
    )(hp, aff2, Wn1b, r1(bn1b))

    k4 = functools.partial(
        pl.kernel,
        mesh=mesh,
        out_type=jax.ShapeDtypeStruct((NC, NP, W2), f32),
        scratch_types=[
            pltpu.VMEM((NCHUNK, K), jnp.int32),
            pltpu.VMEM((K, W2), f32),
            pltpu.VMEM((K, W2), f32),
            pltpu.VMEM((ZR, W2), f32),
            pltpu.VMEM_SHARED((NP, W2), f32),
            pltpu.SemaphoreType.DMA,
            pltpu.SemaphoreType.DMA,
            pltpu.SemaphoreType.DMA,
            pltpu.SemaphoreType.DMA,
        ],
    )(_k4_body)
    acc = k4(h3p, col3)

    un = pl.pallas_call(
        _k5_body,
        out_shape=jax.ShapeDtypeStruct((G, OUT), f32),
    )(xbn, acc[0], acc[1], batch.astype(jnp.int32).reshape(N, 1),
      Wn2a[:D], Wn2a[D:], r1(bn2a), r1(gn2), r1(bbn2),
      Wn2b, r1(bn2b), Wg1, r1(bg1), r1(gg1), r1(bbg1), Wg2, r1(bg2))
    return un
